# baseline (speedup 1.0000x reference)
import functools

import jax
import jax.numpy as jnp
from jax import lax
from jax.experimental import pallas as pl
from jax.experimental.pallas import tpu as pltpu
from jax.experimental.pallas import tpu_sc as plsc

NC = 2
NS = 16
NW = NC * NS
LANES = 16
CHUNK = 80
G_SEGMENTS = 128


def _round_up(a, b):
  return (a + b - 1) // b * b


def _zero_acc(zero_v, acc_sh, zsem, r0, rpt, D):
  zr = zero_v.shape[0]
  zero = jnp.zeros((LANES,), jnp.float32)
  for r in range(zr):
    for c in range(D // LANES):
      zero_v[r, pl.ds(c * LANES, LANES)] = zero
  nz = rpt // zr
  def fire(i, carry):
    pltpu.async_copy(zero_v, acc_sh.at[pl.ds(r0 + i * zr, zr)], zsem)
    return carry
  lax.fori_loop(0, nz, fire, 0)
  def drain(i, carry):
    pltpu.make_async_copy(zero_v, acc_sh.at[pl.ds(r0 + i * zr, zr)],
                          zsem).wait()
    return carry
  lax.fori_loop(0, nz, drain, 0)


def _make_deg_kernel(Ep, Np):
  e_per_w = Ep // NW
  n_chunks = e_per_w // CHUNK
  rpt = Np // NS
  mesh = plsc.VectorSubcoreMesh(core_axis_name="c", subcore_axis_name="s")

  @functools.partial(
      pl.kernel,
      out_type=jax.ShapeDtypeStruct((NC * Np, LANES), jnp.float32),
      mesh=mesh,
      scratch_types=[
          pltpu.VMEM((n_chunks, CHUNK), jnp.int32),
          pltpu.VMEM((CHUNK, LANES), jnp.float32),
          pltpu.VMEM((16, LANES), jnp.float32),
          pltpu.VMEM_SHARED((Np, LANES), jnp.float32),
          pltpu.SemaphoreType.DMA,
          pltpu.SemaphoreType.DMA,
          pltpu.SemaphoreType.DMA,
          pltpu.SemaphoreType.DMA,
      ],
      compiler_params=pltpu.CompilerParams(use_tc_tiling_on_sc=True),
  )
  def k(dst_hbm, out_hbm, ed_all, ones_v, zero_v, acc_sh, s0, s1, zsem, psem):
    cid = lax.axis_index("c")
    sid = lax.axis_index("s")
    wid = sid * NC + cid
    ebase = wid * e_per_w
    one = jnp.ones((LANES,), jnp.float32)
    for r in range(CHUNK):
      ones_v[r, :] = one
    def pfire(c, carry):
      pltpu.async_copy(dst_hbm.at[pl.ds(ebase + c * CHUNK, CHUNK)],
                       ed_all.at[c], psem)
      return carry
    lax.fori_loop(0, n_chunks, pfire, 0)
    def pdrain(c, carry):
      pltpu.make_async_copy(dst_hbm.at[pl.ds(ebase, CHUNK)],
                            ed_all.at[c], psem).wait()
      return carry
    lax.fori_loop(0, n_chunks, pdrain, 0)
    r0 = sid * rpt
    _zero_acc(zero_v, acc_sh, zsem, r0, rpt, LANES)
    plsc.subcore_barrier()
    pltpu.async_copy(ones_v, acc_sh.at[ed_all.at[0]], s0, add=True)
    pltpu.async_copy(ones_v, acc_sh.at[ed_all.at[1]], s1, add=True)
    def body(j2, carry):
      j = j2 * 2
      pltpu.make_async_copy(ones_v, acc_sh.at[ed_all.at[j]], s0).wait()
      pltpu.async_copy(ones_v, acc_sh.at[ed_all.at[j + 2]], s0, add=True)
      pltpu.make_async_copy(ones_v, acc_sh.at[ed_all.at[j + 1]], s1).wait()
      pltpu.async_copy(ones_v, acc_sh.at[ed_all.at[j + 3]], s1, add=True)
      return carry
    lax.fori_loop(0, n_chunks // 2 - 1, body, 0)
    pltpu.make_async_copy(ones_v, acc_sh.at[ed_all.at[n_chunks - 2]],
                          s0).wait()
    pltpu.make_async_copy(ones_v, acc_sh.at[ed_all.at[n_chunks - 1]],
                          s1).wait()
    plsc.subcore_barrier()
    pltpu.sync_copy(acc_sh.at[pl.ds(r0, rpt)],
                    out_hbm.at[pl.ds(cid * Np + r0, rpt)])

  return k


def _make_scatter_kernel(Ep, Np, D, chunk):
  e_per_w = Ep // NW
  n_chunks = e_per_w // chunk
  rpt = Np // NS
  mesh = plsc.VectorSubcoreMesh(core_axis_name="c", subcore_axis_name="s")

  @functools.partial(
      pl.kernel,
      out_type=jax.ShapeDtypeStruct((NC * Np, 128), jnp.float32),
      mesh=mesh,
      scratch_types=[
          pltpu.VMEM((8, 2, chunk), jnp.int32),
          pltpu.VMEM((2, chunk, D), jnp.float32),
          pltpu.VMEM((8, D), jnp.float32),
          pltpu.VMEM_SHARED((Np, D), jnp.float32),
          [pltpu.SemaphoreType.DMA] * 8,
          [pltpu.SemaphoreType.DMA] * 2,
          [pltpu.SemaphoreType.DMA] * 2,
          pltpu.SemaphoreType.DMA,
      ],
      compiler_params=pltpu.CompilerParams(use_tc_tiling_on_sc=False),
  )
  def k(src_hbm, dst_hbm, vals_hbm, out_hbm,
        idx, rows, zero_v, acc_sh, isem, gsem, ssem, zsem):
    cid = lax.axis_index("c")
    sid = lax.axis_index("s")
    wid = sid * NC + cid
    ebase = wid * e_per_w

    def idx_load(c, b):
      off = pl.multiple_of(ebase + c * chunk, chunk)
      pltpu.async_copy(src_hbm.at[pl.ds(off, chunk)], idx.at[b, 0], isem[b])
      pltpu.async_copy(dst_hbm.at[pl.ds(off, chunk)], idx.at[b, 1], isem[b])

    def idx_wait(b):
      pltpu.make_async_copy(src_hbm.at[pl.ds(0, chunk)],
                            idx.at[b, 0], isem[b]).wait()
      pltpu.make_async_copy(src_hbm.at[pl.ds(0, chunk)],
                            idx.at[b, 1], isem[b]).wait()

    def gather(c_ref, rb):
      pltpu.async_copy(vals_hbm.at[c_ref], rows.at[rb], gsem[rb])

    def gather_wait(c_ref, rb):
      pltpu.make_async_copy(vals_hbm.at[c_ref], rows.at[rb],
                            gsem[rb]).wait()

    def scatter(rb, d_ref):
      pltpu.async_copy(rows.at[rb], acc_sh.at[d_ref], ssem[rb], add=True)

    def scatter_wait(rb, d_ref):
      pltpu.make_async_copy(rows.at[rb], acc_sh.at[d_ref], ssem[rb]).wait()

    for b in range(8):
      idx_load(b, b)
    idx_wait(0)
    gather(idx.at[0, 0], 0)
    r0 = sid * rpt
    _zero_acc(zero_v, acc_sh, zsem, r0, rpt, D)
    plsc.subcore_barrier()

    def body(q, carry):
      j0 = q * 8
      for k8 in range(8):
        j = j0 + k8
        rb = k8 % 2
        ib = k8
        rb1 = (k8 + 1) % 2
        ib1 = (k8 + 1) % 8
        ibf = (k8 + 7) % 8
        gather_wait(idx.at[ib, 0], rb)
        scatter(rb, idx.at[ib, 1])
        @pl.when(j + 1 < n_chunks)
        def _():
          @pl.when(j >= 1)
          def _():
            scatter_wait(rb1, idx.at[ibf, 1])
          idx_wait(ib1)
          gather(idx.at[ib1, 0], rb1)
        @pl.when(jnp.logical_and(j >= 1, j + 7 < n_chunks))
        def _():
          idx_load(j + 7, ibf)
      return carry

    lax.fori_loop(0, n_chunks // 8, body, 0)
    for rb in range(2):
      scatter_wait(rb, idx.at[rb, 1])
    plsc.subcore_barrier()
    pltpu.sync_copy(acc_sh.at[pl.ds(r0, rpt)],
                    out_hbm.at[pl.ds(cid * Np + r0, rpt), pl.ds(0, D)])

  return k


_RB = 512
_RBC = 512


def _dis_from(d0, d1):
  deg = d0[:, 0:1] + d1[:, 0:1] + 1.0
  return lax.rsqrt(deg)


def _tc_prep(x_p, W1, degp):
  Np, Din = x_p.shape
  Dh = W1.shape[1]
  nb = Np // _RB

  def body(x_ref, w_ref, d0_ref, d1_ref, o_ref):
    dis = _dis_from(d0_ref[...], d1_ref[...])
    o_ref[...] = jnp.dot(x_ref[...], w_ref[...],
                         preferred_element_type=jnp.float32) * dis

  return pl.pallas_call(
      body,
      grid=(nb,),
      in_specs=[
          pl.BlockSpec((_RB, Din), lambda i: (i, 0)),
          pl.BlockSpec((Din, Dh), lambda i: (0, 0)),
          pl.BlockSpec((_RB, LANES), lambda i: (i, 0)),
          pl.BlockSpec((_RB, LANES), lambda i: (i + nb, 0)),
      ],
      out_specs=pl.BlockSpec((_RB, Dh), lambda i: (i, 0)),
      out_shape=jax.ShapeDtypeStruct((Np, Dh), jnp.float32),
  )(x_p, W1, degp, degp)


def _tc_mid(agg1, m1, degp, W2, b1):
  Np, Dh = m1.shape
  Do = W2.shape[1]
  nb = Np // _RB

  def body(a0_ref, a1_ref, m1_ref, d0_ref, d1_ref, w_ref, b_ref, o_ref):
    dis = _dis_from(d0_ref[...], d1_ref[...])
    h = jnp.maximum(
        (a0_ref[...] + a1_ref[...] + m1_ref[...]) * dis + b_ref[...], 0.0)
    o_ref[...] = jnp.dot(h, w_ref[...],
                         preferred_element_type=jnp.float32) * dis

  return pl.pallas_call(
      body,
      grid=(nb,),
      in_specs=[
          pl.BlockSpec((_RB, Dh), lambda i: (i, 0)),
          pl.BlockSpec((_RB, Dh), lambda i: (i + nb, 0)),
          pl.BlockSpec((_RB, Dh), lambda i: (i, 0)),
          pl.BlockSpec((_RB, LANES), lambda i: (i, 0)),
          pl.BlockSpec((_RB, LANES), lambda i: (i + nb, 0)),
          pl.BlockSpec((Dh, Do), lambda i: (0, 0)),
          pl.BlockSpec((1, Dh), lambda i: (0, 0)),
      ],
      out_specs=pl.BlockSpec((_RB, Do), lambda i: (i, 0)),
      out_shape=jax.ShapeDtypeStruct((Np, Do), jnp.float32),
  )(agg1, agg1, m1, degp, degp, W2, b1)


def _tc_final(agg2, m2, degp, b2, batch3):
  Np, Do = m2.shape
  nsteps = Np // _RBC
  nb = Np // _RBC

  def body(a0_ref, a1_ref, m2_ref, d0_ref, d1_ref, b_ref, bt_ref,
           o_ref, sums, cnts):
    i = pl.program_id(0)

    @pl.when(i == 0)
    def _():
      sums[...] = jnp.zeros_like(sums)
      cnts[...] = jnp.zeros_like(cnts)

    dis = _dis_from(d0_ref[...], d1_ref[...])
    h = jnp.maximum(
        (a0_ref[:, :Do] + a1_ref[:, :Do] + m2_ref[...]) * dis + b_ref[...],
        0.0)
    bt = bt_ref[0]
    mask = (bt == lax.broadcasted_iota(jnp.int32, (G_SEGMENTS, _RBC), 0))
    maskf = mask.astype(jnp.float32)
    sums[...] += jnp.dot(maskf, h,
                         preferred_element_type=jnp.float32)
    cnts[...] += jnp.broadcast_to(
        jnp.sum(maskf, axis=1, keepdims=True), cnts.shape)

    @pl.when(i == nsteps - 1)
    def _():
      o_ref[...] = sums[...] / jnp.maximum(cnts[...], 1.0)

  return pl.pallas_call(
      body,
      grid=(nsteps,),
      in_specs=[
          pl.BlockSpec((_RBC, 128), lambda i: (i, 0)),
          pl.BlockSpec((_RBC, 128), lambda i: (i + nb, 0)),
          pl.BlockSpec((_RBC, Do), lambda i: (i, 0)),
          pl.BlockSpec((_RBC, LANES), lambda i: (i, 0)),
          pl.BlockSpec((_RBC, LANES), lambda i: (i + Np // _RBC, 0)),
          pl.BlockSpec((1, Do), lambda i: (0, 0)),
          pl.BlockSpec((1, 1, _RBC), lambda i: (i, 0, 0)),
      ],
      out_specs=pl.BlockSpec((G_SEGMENTS, Do), lambda i: (0, 0)),
      out_shape=jax.ShapeDtypeStruct((G_SEGMENTS, Do), jnp.float32),
      scratch_shapes=[
          pltpu.VMEM((G_SEGMENTS, Do), jnp.float32),
          pltpu.VMEM((G_SEGMENTS, Do), jnp.float32),
      ],
  )(agg2, agg2, m2, degp, degp, b2, batch3)


def kernel(x, edge_index, batch, W1, b1, W2, b2):
  N, Din = x.shape
  E = edge_index.shape[1]
  Dh = W1.shape[1]
  Do = W2.shape[1]

  Np = _round_up(N + 1, NS * 16)
  Ep = _round_up(E, NW * CHUNK * 8)
  pad_rows = Np - N
  pad_e = Ep - E

  pad_idx = N + (jnp.arange(pad_e, dtype=jnp.int32) % pad_rows)
  src_p = jnp.concatenate([edge_index[0], pad_idx])
  dst_p = jnp.concatenate([edge_index[1], pad_idx])
  x_p = jnp.pad(x, ((0, pad_rows), (0, 0)))
  batch3 = jnp.pad(batch, (0, pad_rows),
                   constant_values=G_SEGMENTS).reshape(Np // _RBC, 1, _RBC)

  degp = _make_deg_kernel(Ep, Np)(dst_p)
  m1 = _tc_prep(x_p, W1, degp)
  agg1 = _make_scatter_kernel(Ep, Np, Dh, 128)(src_p, dst_p, m1)
  m2 = _tc_mid(agg1, m1, degp, W2, b1.reshape(1, Dh))
  agg2 = _make_scatter_kernel(Ep, Np, Do, 128)(src_p, dst_p, m2)
  return _tc_final(agg2, m2, degp, b2.reshape(1, Do), batch3)


# speedup vs baseline: 1.1019x; 1.1019x over previous
import functools

import jax
import jax.numpy as jnp
from jax import lax
from jax.experimental import pallas as pl
from jax.experimental.pallas import tpu as pltpu
from jax.experimental.pallas import tpu_sc as plsc

NC = 2
NS = 16
NW = NC * NS
LANES = 16
CHUNK = 80
G_SEGMENTS = 128


def _round_up(a, b):
  return (a + b - 1) // b * b


def _zero_acc(zero_v, acc_sh, zsem, r0, rpt, D):
  zr = zero_v.shape[0]
  zero = jnp.zeros((LANES,), jnp.float32)
  for r in range(zr):
    for c in range(D // LANES):
      zero_v[r, pl.ds(c * LANES, LANES)] = zero
  nz = rpt // zr
  def fire(i, carry):
    pltpu.async_copy(zero_v, acc_sh.at[pl.ds(r0 + i * zr, zr)], zsem)
    return carry
  lax.fori_loop(0, nz, fire, 0)
  def drain(i, carry):
    pltpu.make_async_copy(zero_v, acc_sh.at[pl.ds(r0 + i * zr, zr)],
                          zsem).wait()
    return carry
  lax.fori_loop(0, nz, drain, 0)


def _make_deg_kernel(Ep, Np):
  e_per_w = Ep // NW
  n_chunks = e_per_w // CHUNK
  rpt = Np // NS
  mesh = plsc.VectorSubcoreMesh(core_axis_name="c", subcore_axis_name="s")

  @functools.partial(
      pl.kernel,
      out_type=jax.ShapeDtypeStruct((NC * Np, LANES), jnp.float32),
      mesh=mesh,
      scratch_types=[
          pltpu.VMEM((n_chunks, CHUNK), jnp.int32),
          pltpu.VMEM((CHUNK, LANES), jnp.float32),
          pltpu.VMEM((16, LANES), jnp.float32),
          pltpu.VMEM_SHARED((Np, LANES), jnp.float32),
          pltpu.SemaphoreType.DMA,
          pltpu.SemaphoreType.DMA,
          pltpu.SemaphoreType.DMA,
          pltpu.SemaphoreType.DMA,
      ],
      compiler_params=pltpu.CompilerParams(use_tc_tiling_on_sc=True),
  )
  def k(dst_hbm, out_hbm, ed_all, ones_v, zero_v, acc_sh, s0, s1, zsem, psem):
    cid = lax.axis_index("c")
    sid = lax.axis_index("s")
    wid = sid * NC + cid
    ebase = wid * e_per_w
    one = jnp.ones((LANES,), jnp.float32)
    for r in range(CHUNK):
      ones_v[r, :] = one
    def pfire(c, carry):
      pltpu.async_copy(dst_hbm.at[pl.ds(ebase + c * CHUNK, CHUNK)],
                       ed_all.at[c], psem)
      return carry
    lax.fori_loop(0, n_chunks, pfire, 0)
    def pdrain(c, carry):
      pltpu.make_async_copy(dst_hbm.at[pl.ds(ebase, CHUNK)],
                            ed_all.at[c], psem).wait()
      return carry
    lax.fori_loop(0, n_chunks, pdrain, 0)
    r0 = sid * rpt
    _zero_acc(zero_v, acc_sh, zsem, r0, rpt, LANES)
    plsc.subcore_barrier()
    pltpu.async_copy(ones_v, acc_sh.at[ed_all.at[0]], s0, add=True)
    pltpu.async_copy(ones_v, acc_sh.at[ed_all.at[1]], s1, add=True)
    def body(j2, carry):
      j = j2 * 2
      pltpu.make_async_copy(ones_v, acc_sh.at[ed_all.at[j]], s0).wait()
      pltpu.async_copy(ones_v, acc_sh.at[ed_all.at[j + 2]], s0, add=True)
      pltpu.make_async_copy(ones_v, acc_sh.at[ed_all.at[j + 1]], s1).wait()
      pltpu.async_copy(ones_v, acc_sh.at[ed_all.at[j + 3]], s1, add=True)
      return carry
    lax.fori_loop(0, n_chunks // 2 - 1, body, 0)
    pltpu.make_async_copy(ones_v, acc_sh.at[ed_all.at[n_chunks - 2]],
                          s0).wait()
    pltpu.make_async_copy(ones_v, acc_sh.at[ed_all.at[n_chunks - 1]],
                          s1).wait()
    plsc.subcore_barrier()
    pltpu.sync_copy(acc_sh.at[pl.ds(r0, rpt)],
                    out_hbm.at[pl.ds(cid * Np + r0, rpt)])

  return k


def _make_scatter_kernel(Ep, Np, D, chunk):
  e_per_w = Ep // NW
  n_chunks = e_per_w // chunk
  rpt = Np // NS
  mesh = plsc.VectorSubcoreMesh(core_axis_name="c", subcore_axis_name="s")

  @functools.partial(
      pl.kernel,
      out_type=jax.ShapeDtypeStruct((NC * Np, 128), jnp.float32),
      mesh=mesh,
      scratch_types=[
          pltpu.VMEM((8, 2, chunk), jnp.int32),
          pltpu.VMEM((4, chunk, D), jnp.float32),
          pltpu.VMEM((8, D), jnp.float32),
          pltpu.VMEM_SHARED((Np, D), jnp.float32),
          [pltpu.SemaphoreType.DMA] * 8,
          [pltpu.SemaphoreType.DMA] * 4,
          [pltpu.SemaphoreType.DMA] * 4,
          pltpu.SemaphoreType.DMA,
      ],
      compiler_params=pltpu.CompilerParams(use_tc_tiling_on_sc=False),
  )
  def k(src_hbm, dst_hbm, vals_hbm, out_hbm,
        idx, rows, zero_v, acc_sh, isem, gsem, ssem, zsem):
    cid = lax.axis_index("c")
    sid = lax.axis_index("s")
    wid = sid * NC + cid
    ebase = wid * e_per_w

    def idx_load(c, b):
      off = pl.multiple_of(ebase + c * chunk, chunk)
      pltpu.async_copy(src_hbm.at[pl.ds(off, chunk)], idx.at[b, 0], isem[b])
      pltpu.async_copy(dst_hbm.at[pl.ds(off, chunk)], idx.at[b, 1], isem[b])

    def idx_wait(b):
      pltpu.make_async_copy(src_hbm.at[pl.ds(0, chunk)],
                            idx.at[b, 0], isem[b]).wait()
      pltpu.make_async_copy(src_hbm.at[pl.ds(0, chunk)],
                            idx.at[b, 1], isem[b]).wait()

    def gather(c_ref, rb):
      pltpu.async_copy(vals_hbm.at[c_ref], rows.at[rb], gsem[rb])

    def gather_wait(c_ref, rb):
      pltpu.make_async_copy(vals_hbm.at[c_ref], rows.at[rb],
                            gsem[rb]).wait()

    def scatter(rb, d_ref):
      pltpu.async_copy(rows.at[rb], acc_sh.at[d_ref], ssem[rb], add=True)

    def scatter_wait(rb, d_ref):
      pltpu.make_async_copy(rows.at[rb], acc_sh.at[d_ref], ssem[rb]).wait()

    for b in range(8):
      idx_load(b, b)
    idx_wait(0)
    gather(idx.at[0, 0], 0)
    idx_wait(1)
    gather(idx.at[1, 0], 1)
    r0 = sid * rpt
    _zero_acc(zero_v, acc_sh, zsem, r0, rpt, D)
    plsc.subcore_barrier()

    def body(q, carry):
      j0 = q * 8
      for k8 in range(8):
        j = j0 + k8
        rb = k8 % 4
        ib = k8
        rb2 = (k8 + 2) % 4
        ib2 = (k8 + 2) % 8
        ibf = (k8 + 6) % 8
        gather_wait(idx.at[ib, 0], rb)
        scatter(rb, idx.at[ib, 1])
        @pl.when(j + 2 < n_chunks)
        def _():
          @pl.when(j >= 2)
          def _():
            scatter_wait(rb2, idx.at[ibf, 1])
          idx_wait(ib2)
          gather(idx.at[ib2, 0], rb2)
        @pl.when(jnp.logical_and(j >= 2, j + 6 < n_chunks))
        def _():
          idx_load(j + 6, ibf)
      return carry

    lax.fori_loop(0, n_chunks // 8, body, 0)
    for rb in range(4):
      scatter_wait(rb, idx.at[rb, 1])
    plsc.subcore_barrier()
    pltpu.sync_copy(acc_sh.at[pl.ds(r0, rpt)],
                    out_hbm.at[pl.ds(cid * Np + r0, rpt), pl.ds(0, D)])

  return k


_RB = 512
_RBC = 512


def _dis_from(d0, d1):
  deg = d0[:, 0:1] + d1[:, 0:1] + 1.0
  return lax.rsqrt(deg)


def _tc_prep(x_p, W1, degp):
  Np, Din = x_p.shape
  Dh = W1.shape[1]
  nb = Np // _RB

  def body(x_ref, w_ref, d0_ref, d1_ref, o_ref):
    dis = _dis_from(d0_ref[...], d1_ref[...])
    o_ref[...] = jnp.dot(x_ref[...], w_ref[...],
                         preferred_element_type=jnp.float32) * dis

  return pl.pallas_call(
      body,
      grid=(nb,),
      in_specs=[
          pl.BlockSpec((_RB, Din), lambda i: (i, 0)),
          pl.BlockSpec((Din, Dh), lambda i: (0, 0)),
          pl.BlockSpec((_RB, LANES), lambda i: (i, 0)),
          pl.BlockSpec((_RB, LANES), lambda i: (i + nb, 0)),
      ],
      out_specs=pl.BlockSpec((_RB, Dh), lambda i: (i, 0)),
      out_shape=jax.ShapeDtypeStruct((Np, Dh), jnp.float32),
  )(x_p, W1, degp, degp)


def _tc_mid(agg1, m1, degp, W2, b1):
  Np, Dh = m1.shape
  Do = W2.shape[1]
  nb = Np // _RB

  def body(a0_ref, a1_ref, m1_ref, d0_ref, d1_ref, w_ref, b_ref, o_ref):
    dis = _dis_from(d0_ref[...], d1_ref[...])
    h = jnp.maximum(
        (a0_ref[...] + a1_ref[...] + m1_ref[...]) * dis + b_ref[...], 0.0)
    o_ref[...] = jnp.dot(h, w_ref[...],
                         preferred_element_type=jnp.float32) * dis

  return pl.pallas_call(
      body,
      grid=(nb,),
      in_specs=[
          pl.BlockSpec((_RB, Dh), lambda i: (i, 0)),
          pl.BlockSpec((_RB, Dh), lambda i: (i + nb, 0)),
          pl.BlockSpec((_RB, Dh), lambda i: (i, 0)),
          pl.BlockSpec((_RB, LANES), lambda i: (i, 0)),
          pl.BlockSpec((_RB, LANES), lambda i: (i + nb, 0)),
          pl.BlockSpec((Dh, Do), lambda i: (0, 0)),
          pl.BlockSpec((1, Dh), lambda i: (0, 0)),
      ],
      out_specs=pl.BlockSpec((_RB, Do), lambda i: (i, 0)),
      out_shape=jax.ShapeDtypeStruct((Np, Do), jnp.float32),
  )(agg1, agg1, m1, degp, degp, W2, b1)


def _tc_final(agg2, m2, degp, b2, batch3):
  Np, Do = m2.shape
  nsteps = Np // _RBC
  nb = Np // _RBC

  def body(a0_ref, a1_ref, m2_ref, d0_ref, d1_ref, b_ref, bt_ref,
           o_ref, sums, cnts):
    i = pl.program_id(0)

    @pl.when(i == 0)
    def _():
      sums[...] = jnp.zeros_like(sums)
      cnts[...] = jnp.zeros_like(cnts)

    dis = _dis_from(d0_ref[...], d1_ref[...])
    h = jnp.maximum(
        (a0_ref[:, :Do] + a1_ref[:, :Do] + m2_ref[...]) * dis + b_ref[...],
        0.0)
    bt = bt_ref[0]
    mask = (bt == lax.broadcasted_iota(jnp.int32, (G_SEGMENTS, _RBC), 0))
    maskf = mask.astype(jnp.float32)
    sums[...] += jnp.dot(maskf, h,
                         preferred_element_type=jnp.float32)
    cnts[...] += jnp.broadcast_to(
        jnp.sum(maskf, axis=1, keepdims=True), cnts.shape)

    @pl.when(i == nsteps - 1)
    def _():
      o_ref[...] = sums[...] / jnp.maximum(cnts[...], 1.0)

  return pl.pallas_call(
      body,
      grid=(nsteps,),
      in_specs=[
          pl.BlockSpec((_RBC, 128), lambda i: (i, 0)),
          pl.BlockSpec((_RBC, 128), lambda i: (i + nb, 0)),
          pl.BlockSpec((_RBC, Do), lambda i: (i, 0)),
          pl.BlockSpec((_RBC, LANES), lambda i: (i, 0)),
          pl.BlockSpec((_RBC, LANES), lambda i: (i + Np // _RBC, 0)),
          pl.BlockSpec((1, Do), lambda i: (0, 0)),
          pl.BlockSpec((1, 1, _RBC), lambda i: (i, 0, 0)),
      ],
      out_specs=pl.BlockSpec((G_SEGMENTS, Do), lambda i: (0, 0)),
      out_shape=jax.ShapeDtypeStruct((G_SEGMENTS, Do), jnp.float32),
      scratch_shapes=[
          pltpu.VMEM((G_SEGMENTS, Do), jnp.float32),
          pltpu.VMEM((G_SEGMENTS, Do), jnp.float32),
      ],
  )(agg2, agg2, m2, degp, degp, b2, batch3)


def kernel(x, edge_index, batch, W1, b1, W2, b2):
  N, Din = x.shape
  E = edge_index.shape[1]
  Dh = W1.shape[1]
  Do = W2.shape[1]

  Np = _round_up(N + 1, NS * 16)
  Ep = _round_up(E, NW * CHUNK * 8)
  pad_rows = Np - N
  pad_e = Ep - E

  pad_idx = N + (jnp.arange(pad_e, dtype=jnp.int32) % pad_rows)
  src_p = jnp.concatenate([edge_index[0], pad_idx])
  dst_p = jnp.concatenate([edge_index[1], pad_idx])
  x_p = jnp.pad(x, ((0, pad_rows), (0, 0)))
  batch3 = jnp.pad(batch, (0, pad_rows),
                   constant_values=G_SEGMENTS).reshape(Np // _RBC, 1, _RBC)

  degp = _make_deg_kernel(Ep, Np)(dst_p)
  m1 = _tc_prep(x_p, W1, degp)
  agg1 = _make_scatter_kernel(Ep, Np, Dh, CHUNK)(src_p, dst_p, m1)
  m2 = _tc_mid(agg1, m1, degp, W2, b1.reshape(1, Dh))
  agg2 = _make_scatter_kernel(Ep, Np, Do, CHUNK)(src_p, dst_p, m2)
  return _tc_final(agg2, m2, degp, b2.reshape(1, Do), batch3)


# speedup vs baseline: 1.1372x; 1.0321x over previous
import functools

import jax
import jax.numpy as jnp
from jax import lax
from jax.experimental import pallas as pl
from jax.experimental.pallas import tpu as pltpu
from jax.experimental.pallas import tpu_sc as plsc

NC = 2
NS = 16
NW = NC * NS
LANES = 16
CHUNK = 80
G_SEGMENTS = 128


def _round_up(a, b):
  return (a + b - 1) // b * b


def _zero_acc(zero_v, acc_sh, zsem, r0, rpt, D):
  zr = zero_v.shape[0]
  zero = jnp.zeros((LANES,), jnp.float32)
  for r in range(zr):
    for c in range(D // LANES):
      zero_v[r, pl.ds(c * LANES, LANES)] = zero
  nz = rpt // zr
  def fire(i, carry):
    pltpu.async_copy(zero_v, acc_sh.at[pl.ds(r0 + i * zr, zr)], zsem)
    return carry
  lax.fori_loop(0, nz, fire, 0)
  def drain(i, carry):
    pltpu.make_async_copy(zero_v, acc_sh.at[pl.ds(r0 + i * zr, zr)],
                          zsem).wait()
    return carry
  lax.fori_loop(0, nz, drain, 0)


def _make_deg_kernel(Ep, Np):
  e_per_w = Ep // NW
  n_chunks = e_per_w // CHUNK
  rpt = Np // NS
  mesh = plsc.VectorSubcoreMesh(core_axis_name="c", subcore_axis_name="s")

  @functools.partial(
      pl.kernel,
      out_type=jax.ShapeDtypeStruct((NC * Np, LANES), jnp.float32),
      mesh=mesh,
      scratch_types=[
          pltpu.VMEM((n_chunks, CHUNK), jnp.int32),
          pltpu.VMEM((CHUNK, LANES), jnp.float32),
          pltpu.VMEM((16, LANES), jnp.float32),
          pltpu.VMEM_SHARED((Np, LANES), jnp.float32),
          pltpu.SemaphoreType.DMA,
          pltpu.SemaphoreType.DMA,
          pltpu.SemaphoreType.DMA,
          pltpu.SemaphoreType.DMA,
      ],
      compiler_params=pltpu.CompilerParams(use_tc_tiling_on_sc=True),
  )
  def k(dst_hbm, out_hbm, ed_all, ones_v, zero_v, acc_sh, s0, s1, zsem, psem):
    cid = lax.axis_index("c")
    sid = lax.axis_index("s")
    wid = sid * NC + cid
    ebase = wid * e_per_w
    one = jnp.ones((LANES,), jnp.float32)
    for r in range(CHUNK):
      ones_v[r, :] = one
    def pfire(c, carry):
      pltpu.async_copy(dst_hbm.at[pl.ds(ebase + c * CHUNK, CHUNK)],
                       ed_all.at[c], psem)
      return carry
    lax.fori_loop(0, n_chunks, pfire, 0)
    def pdrain(c, carry):
      pltpu.make_async_copy(dst_hbm.at[pl.ds(ebase, CHUNK)],
                            ed_all.at[c], psem).wait()
      return carry
    lax.fori_loop(0, n_chunks, pdrain, 0)
    r0 = sid * rpt
    _zero_acc(zero_v, acc_sh, zsem, r0, rpt, LANES)
    plsc.subcore_barrier()
    pltpu.async_copy(ones_v, acc_sh.at[ed_all.at[0]], s0, add=True)
    pltpu.async_copy(ones_v, acc_sh.at[ed_all.at[1]], s1, add=True)
    def body(j2, carry):
      j = j2 * 2
      pltpu.make_async_copy(ones_v, acc_sh.at[ed_all.at[j]], s0).wait()
      pltpu.async_copy(ones_v, acc_sh.at[ed_all.at[j + 2]], s0, add=True)
      pltpu.make_async_copy(ones_v, acc_sh.at[ed_all.at[j + 1]], s1).wait()
      pltpu.async_copy(ones_v, acc_sh.at[ed_all.at[j + 3]], s1, add=True)
      return carry
    lax.fori_loop(0, n_chunks // 2 - 1, body, 0)
    pltpu.make_async_copy(ones_v, acc_sh.at[ed_all.at[n_chunks - 2]],
                          s0).wait()
    pltpu.make_async_copy(ones_v, acc_sh.at[ed_all.at[n_chunks - 1]],
                          s1).wait()
    plsc.subcore_barrier()
    pltpu.sync_copy(acc_sh.at[pl.ds(r0, rpt)],
                    out_hbm.at[pl.ds(cid * Np + r0, rpt)])

  return k


def _make_scatter_kernel(Ep, Np, D, chunk):
  e_per_w = Ep // NW
  n_chunks = e_per_w // chunk
  rpt = Np // NS
  mesh = plsc.VectorSubcoreMesh(core_axis_name="c", subcore_axis_name="s")

  @functools.partial(
      pl.kernel,
      out_type=jax.ShapeDtypeStruct((NC * Np, 128), jnp.float32),
      mesh=mesh,
      scratch_types=[
          pltpu.VMEM((8, 2, chunk), jnp.int32),
          pltpu.VMEM((4, chunk, D), jnp.float32),
          pltpu.VMEM((8, D), jnp.float32),
          pltpu.VMEM_SHARED((Np, D), jnp.float32),
          [pltpu.SemaphoreType.DMA] * 8,
          [pltpu.SemaphoreType.DMA] * 4,
          [pltpu.SemaphoreType.DMA] * 4,
          pltpu.SemaphoreType.DMA,
      ],
      compiler_params=pltpu.CompilerParams(use_tc_tiling_on_sc=False),
  )
  def k(src_hbm, dst_hbm, vals_hbm, out_hbm,
        idx, rows, zero_v, acc_sh, isem, gsem, ssem, zsem):
    cid = lax.axis_index("c")
    sid = lax.axis_index("s")
    wid = sid * NC + cid
    ebase = wid * e_per_w

    def idx_load(c, b):
      off = pl.multiple_of(ebase + c * chunk, chunk)
      pltpu.async_copy(src_hbm.at[pl.ds(off, chunk)], idx.at[b, 0], isem[b])
      pltpu.async_copy(dst_hbm.at[pl.ds(off, chunk)], idx.at[b, 1], isem[b])

    def idx_wait(b):
      pltpu.make_async_copy(src_hbm.at[pl.ds(0, chunk)],
                            idx.at[b, 0], isem[b]).wait()
      pltpu.make_async_copy(src_hbm.at[pl.ds(0, chunk)],
                            idx.at[b, 1], isem[b]).wait()

    def gather(c_ref, rb):
      pltpu.async_copy(vals_hbm.at[c_ref], rows.at[rb], gsem[rb])

    def gather_wait(c_ref, rb):
      pltpu.make_async_copy(vals_hbm.at[c_ref], rows.at[rb],
                            gsem[rb]).wait()

    def scatter(rb, d_ref):
      pltpu.async_copy(rows.at[rb], acc_sh.at[d_ref], ssem[rb], add=True)

    def scatter_wait(rb, d_ref):
      pltpu.make_async_copy(rows.at[rb], acc_sh.at[d_ref], ssem[rb]).wait()

    for b in range(8):
      idx_load(b, b)
    idx_wait(0)
    gather(idx.at[0, 0], 0)
    idx_wait(1)
    gather(idx.at[1, 0], 1)
    r0 = sid * rpt
    _zero_acc(zero_v, acc_sh, zsem, r0, rpt, D)
    plsc.subcore_barrier()

    def body(q, carry):
      j0 = q * 8
      for k8 in range(8):
        j = j0 + k8
        rb = k8 % 4
        ib = k8
        rb2 = (k8 + 2) % 4
        ib2 = (k8 + 2) % 8
        ibf = (k8 + 6) % 8
        gather_wait(idx.at[ib, 0], rb)
        scatter(rb, idx.at[ib, 1])
        @pl.when(j + 2 < n_chunks)
        def _():
          @pl.when(j >= 2)
          def _():
            scatter_wait(rb2, idx.at[ibf, 1])
          idx_wait(ib2)
          gather(idx.at[ib2, 0], rb2)
        @pl.when(jnp.logical_and(j >= 2, j + 6 < n_chunks))
        def _():
          idx_load(j + 6, ibf)
      return carry

    lax.fori_loop(0, n_chunks // 8, body, 0)
    for rb in range(4):
      scatter_wait(rb, idx.at[rb, 1])
    plsc.subcore_barrier()
    pltpu.sync_copy(acc_sh.at[pl.ds(r0, rpt)],
                    out_hbm.at[pl.ds(cid * Np + r0, rpt), pl.ds(0, D)])

  return k


_RB = 512
_RBC = 512


def _dis_from(d0, d1):
  deg = d0[:, 0:1] + d1[:, 0:1] + 1.0
  return lax.rsqrt(deg)


def _tc_prep(x_p, W1, degp):
  Np, Din = x_p.shape
  Dh = W1.shape[1]
  nb = Np // _RB

  def body(x_ref, w_ref, d0_ref, d1_ref, o_ref):
    dis = _dis_from(d0_ref[...], d1_ref[...])
    o_ref[...] = jnp.dot(x_ref[...], w_ref[...],
                         preferred_element_type=jnp.float32) * dis

  return pl.pallas_call(
      body,
      grid=(nb,),
      in_specs=[
          pl.BlockSpec((_RB, Din), lambda i: (i, 0)),
          pl.BlockSpec((Din, Dh), lambda i: (0, 0)),
          pl.BlockSpec((_RB, LANES), lambda i: (i, 0)),
          pl.BlockSpec((_RB, LANES), lambda i: (i + nb, 0)),
      ],
      out_specs=pl.BlockSpec((_RB, Dh), lambda i: (i, 0)),
      out_shape=jax.ShapeDtypeStruct((Np, Dh), jnp.float32),
  )(x_p, W1, degp, degp)


def _tc_mid(agg1, m1, degp, W2, b1):
  Np, Dh = m1.shape
  Do = W2.shape[1]
  nb = Np // _RB

  def body(a0_ref, a1_ref, m1_ref, d0_ref, d1_ref, w_ref, b_ref, o_ref):
    dis = _dis_from(d0_ref[...], d1_ref[...])
    h = jnp.maximum(
        (a0_ref[...] + a1_ref[...] + m1_ref[...]) * dis + b_ref[...], 0.0)
    o_ref[...] = jnp.dot(h, w_ref[...],
                         preferred_element_type=jnp.float32) * dis

  return pl.pallas_call(
      body,
      grid=(nb,),
      in_specs=[
          pl.BlockSpec((_RB, Dh), lambda i: (i, 0)),
          pl.BlockSpec((_RB, Dh), lambda i: (i + nb, 0)),
          pl.BlockSpec((_RB, Dh), lambda i: (i, 0)),
          pl.BlockSpec((_RB, LANES), lambda i: (i, 0)),
          pl.BlockSpec((_RB, LANES), lambda i: (i + nb, 0)),
          pl.BlockSpec((Dh, Do), lambda i: (0, 0)),
          pl.BlockSpec((1, Dh), lambda i: (0, 0)),
      ],
      out_specs=pl.BlockSpec((_RB, Do), lambda i: (i, 0)),
      out_shape=jax.ShapeDtypeStruct((Np, Do), jnp.float32),
  )(agg1, agg1, m1, degp, degp, W2, b1)


def _tc_final(agg2, m2, degp, b2, batch3):
  Np, Do = m2.shape
  nsteps = Np // _RBC
  nb = Np // _RBC

  def body(a0_ref, a1_ref, m2_ref, d0_ref, d1_ref, b_ref, bt_ref,
           o_ref, sums, cnts):
    i = pl.program_id(0)

    @pl.when(i == 0)
    def _():
      sums[...] = jnp.zeros_like(sums)
      cnts[...] = jnp.zeros_like(cnts)

    dis = _dis_from(d0_ref[...], d1_ref[...])
    h = jnp.maximum(
        (a0_ref[:, :Do] + a1_ref[:, :Do] + m2_ref[...]) * dis + b_ref[...],
        0.0)
    bt = bt_ref[0]
    mask = (bt == lax.broadcasted_iota(jnp.int32, (G_SEGMENTS, _RBC), 0))
    maskf = mask.astype(jnp.float32)
    sums[...] += jnp.dot(maskf, h,
                         preferred_element_type=jnp.float32)
    cnts[...] += jnp.broadcast_to(
        jnp.sum(maskf, axis=1, keepdims=True), cnts.shape)

    @pl.when(i == nsteps - 1)
    def _():
      o_ref[...] = sums[...] / jnp.maximum(cnts[...], 1.0)

  return pl.pallas_call(
      body,
      grid=(nsteps,),
      in_specs=[
          pl.BlockSpec((_RBC, 128), lambda i: (i, 0)),
          pl.BlockSpec((_RBC, 128), lambda i: (i + nb, 0)),
          pl.BlockSpec((_RBC, Do), lambda i: (i, 0)),
          pl.BlockSpec((_RBC, LANES), lambda i: (i, 0)),
          pl.BlockSpec((_RBC, LANES), lambda i: (i + Np // _RBC, 0)),
          pl.BlockSpec((1, Do), lambda i: (0, 0)),
          pl.BlockSpec((1, 1, _RBC), lambda i: (i, 0, 0)),
      ],
      out_specs=pl.BlockSpec((G_SEGMENTS, Do), lambda i: (0, 0)),
      out_shape=jax.ShapeDtypeStruct((G_SEGMENTS, Do), jnp.float32),
      scratch_shapes=[
          pltpu.VMEM((G_SEGMENTS, Do), jnp.float32),
          pltpu.VMEM((G_SEGMENTS, Do), jnp.float32),
      ],
  )(agg2, agg2, m2, degp, degp, b2, batch3)


def kernel(x, edge_index, batch, W1, b1, W2, b2):
  N, Din = x.shape
  E = edge_index.shape[1]
  Dh = W1.shape[1]
  Do = W2.shape[1]

  Np = _round_up(N + 1, NS * 16)
  Ep = _round_up(E, NW * CHUNK * 8)
  pad_rows = Np - N
  pad_e = Ep - E

  pad_idx = N + (jnp.arange(pad_e, dtype=jnp.int32) % pad_rows)
  src_p = jnp.concatenate([edge_index[0], pad_idx])
  dst_p = jnp.concatenate([edge_index[1], pad_idx])
  x_p = jnp.pad(x, ((0, pad_rows), (0, 0)))
  batch3 = jnp.pad(batch, (0, pad_rows),
                   constant_values=G_SEGMENTS).reshape(Np // _RBC, 1, _RBC)

  degp = _make_deg_kernel(Ep, Np)(dst_p)
  m1 = _tc_prep(x_p, W1, degp)
  agg1 = _make_scatter_kernel(Ep, Np, Dh, CHUNK)(src_p, dst_p, m1)
  m2 = _tc_mid(agg1, m1, degp, W2, b1.reshape(1, Dh))
  agg2 = _make_scatter_kernel(Ep, Np, Do, 128)(src_p, dst_p, m2)
  return _tc_final(agg2, m2, degp, b2.reshape(1, Do), batch3)
